# baseline (speedup 1.0000x reference)
import functools

import jax
import jax.numpy as jnp
from jax import lax
from jax.experimental import pallas as pl
from jax.experimental.pallas import tpu as pltpu
from jax.experimental.pallas import tpu_sc as plsc

N = 10000
E = 320000
D = 128

NC = 2
NS = 16
NW = NC * NS
NPAD = 10240
SEG = NPAD // NS
EW = E // NW
CH = 80
KC = EW // CH

IP = 5
KP = KC // IP

RB = 1024
GRID = (NPAD + RB - 1) // RB


def _sc_mesh():
    return plsc.VectorSubcoreMesh(
        core_axis_name="c", subcore_axis_name="s", num_cores=NC, num_subcores=NS
    )


def _deg_partials(dst3):
    @functools.partial(
        pl.kernel,
        out_type=(
            jax.ShapeDtypeStruct((NPAD,), jnp.float32),
            jax.ShapeDtypeStruct((NPAD,), jnp.float32),
        ),
        mesh=_sc_mesh(),
        scratch_types=[
            pltpu.VMEM((IP, KP, CH), jnp.int32),
            pltpu.VMEM((CH,), jnp.float32),
            pltpu.VMEM((SEG,), jnp.float32),
            pltpu.VMEM_SHARED((NPAD,), jnp.float32),
            pltpu.SemaphoreType.DMA,
        ],
    )
    def k(dst_hbm, deg0_hbm, deg1_hbm, didx3, ones, zbuf, deg_sh, sem):
        cid = lax.axis_index("c")
        sid = lax.axis_index("s")
        wid = cid * NS + sid
        pltpu.sync_copy(dst_hbm.at[wid], didx3)
        for i in range(CH // 16):
            ones[pl.ds(i * 16, 16)] = jnp.ones((16,), jnp.float32)

        def zfill(i, _):
            zbuf[pl.ds(i * 16, 16)] = jnp.zeros((16,), jnp.float32)
            return 0

        lax.fori_loop(0, SEG // 16, zfill, 0)
        pltpu.sync_copy(zbuf, deg_sh.at[pl.ds(sid * SEG, SEG)])
        plsc.subcore_barrier()

        def issue(kk, _):
            pltpu.async_copy(ones, deg_sh.at[didx3.at[kk // KP, kk % KP]], sem, add=True)
            return 0

        lax.fori_loop(0, KC, issue, 0)

        def drain(kk, _):
            pltpu.make_async_copy(ones, deg_sh.at[didx3.at[kk // KP, kk % KP]], sem).wait()
            return 0

        lax.fori_loop(0, KC, drain, 0)
        plsc.subcore_barrier()

        @pl.when(cid == 0)
        def _():
            pltpu.sync_copy(
                deg_sh.at[pl.ds(sid * SEG, SEG)], deg0_hbm.at[pl.ds(sid * SEG, SEG)]
            )

        @pl.when(cid == 1)
        def _():
            pltpu.sync_copy(
                deg_sh.at[pl.ds(sid * SEG, SEG)], deg1_hbm.at[pl.ds(sid * SEG, SEG)]
            )

    return k(dst3)


def _aggregate(p, src3, dst3):
    @functools.partial(
        pl.kernel,
        out_type=(
            jax.ShapeDtypeStruct((NPAD, D), jnp.float32),
            jax.ShapeDtypeStruct((NPAD, D), jnp.float32),
        ),
        mesh=_sc_mesh(),
        scratch_types=[
            pltpu.VMEM((KP, CH), jnp.int32),
            pltpu.VMEM((KP, CH), jnp.int32),
            [pltpu.VMEM((CH, D), jnp.float32)] * 4,
            pltpu.VMEM_SHARED((NPAD, D), jnp.float32),
            [pltpu.SemaphoreType.DMA] * 4,
            [pltpu.SemaphoreType.DMA] * 4,
            pltpu.SemaphoreType.DMA,
        ],
    )
    def k(p_hbm, src_hbm, dst_hbm, a0_hbm, a1_hbm, sidx2, didx2,
          bufs, acc_sh, gs, ss, zsem):
        cid = lax.axis_index("c")
        sid = lax.axis_index("s")
        wid = cid * NS + sid
        buf0 = bufs[0]

        def zfill(i, _):
            for c in range(D // 16):
                buf0[i, pl.ds(c * 16, 16)] = jnp.zeros((16,), jnp.float32)
            return 0

        lax.fori_loop(0, CH, zfill, 0)
        for i in range(SEG // CH):
            pltpu.async_copy(buf0, acc_sh.at[pl.ds(sid * SEG + i * CH, CH)], zsem)
        for i in range(SEG // CH):
            pltpu.make_async_copy(buf0, acc_sh.at[pl.ds(sid * SEG + i * CH, CH)], zsem).wait()
        plsc.subcore_barrier()

        def issue_g(kk, r):
            pltpu.async_copy(p_hbm.at[sidx2.at[kk]], bufs[r], gs[r])

        def drain_g(kk, r):
            pltpu.make_async_copy(p_hbm.at[sidx2.at[kk]], bufs[r], gs[r]).wait()

        def issue_s(kk, r):
            pltpu.async_copy(bufs[r], acc_sh.at[didx2.at[kk]], ss[r], add=True)

        def drain_s(kk, r):
            pltpu.make_async_copy(bufs[r], acc_sh.at[didx2.at[kk]], ss[r]).wait()

        def full_step(kk, r):
            drain_g(kk, r)
            issue_s(kk, r)
            drain_s(kk - 1, (r + 3) % 4)
            issue_g(kk + 3, (r + 3) % 4)

        for p in range(IP):
            pltpu.sync_copy(src_hbm.at[wid, p], sidx2)
            pltpu.sync_copy(dst_hbm.at[wid, p], didx2)

            issue_g(0, 0)
            issue_g(1, 1)
            issue_g(2, 2)
            drain_g(0, 0)
            issue_s(0, 0)
            issue_g(3, 3)
            full_step(1, 1)
            full_step(2, 2)
            full_step(3, 3)

            def quad(j, _):
                kk = 4 + 4 * j
                full_step(kk, 0)
                full_step(kk + 1, 1)
                full_step(kk + 2, 2)
                full_step(kk + 3, 3)
                return 0

            lax.fori_loop(0, (KP - 9) // 4, quad, 0)

            full_step(KP - 5, 0)
            full_step(KP - 4, 1)
            drain_g(KP - 3, 2)
            issue_s(KP - 3, 2)
            drain_s(KP - 4, 1)
            drain_g(KP - 2, 3)
            issue_s(KP - 2, 3)
            drain_s(KP - 3, 2)
            drain_g(KP - 1, 0)
            issue_s(KP - 1, 0)
            drain_s(KP - 2, 3)
            drain_s(KP - 1, 0)
        plsc.subcore_barrier()

        @pl.when(cid == 0)
        def _():
            pltpu.sync_copy(
                acc_sh.at[pl.ds(sid * SEG, SEG)], a0_hbm.at[pl.ds(sid * SEG, SEG)]
            )

        @pl.when(cid == 1)
        def _():
            pltpu.sync_copy(
                acc_sh.at[pl.ds(sid * SEG, SEG)], a1_hbm.at[pl.ds(sid * SEG, SEG)]
            )

    return k(p, src3, dst3)


def _tc_first(d0, d1, x, W1):
    def body(d0_ref, d1_ref, x_ref, w_ref, p_ref):
        dinv = lax.rsqrt(d0_ref[...] + d1_ref[...] + 1.0)
        z = jnp.dot(
            x_ref[...], w_ref[...],
            preferred_element_type=jnp.float32,
            precision=lax.Precision.HIGHEST,
        )
        p_ref[...] = z * dinv

    return pl.pallas_call(
        body,
        grid=(GRID,),
        in_specs=[
            pl.BlockSpec((RB, 1), lambda i: (i, 0)),
            pl.BlockSpec((RB, 1), lambda i: (i, 0)),
            pl.BlockSpec((RB, D), lambda i: (i, 0)),
            pl.BlockSpec((D, D), lambda i: (0, 0)),
        ],
        out_specs=pl.BlockSpec((RB, D), lambda i: (i, 0)),
        out_shape=jax.ShapeDtypeStruct((N, D), jnp.float32),
    )(d0, d1, x, W1)


def _tc_mid(d0, d1, a0, a1, p1, b1, W2):
    def body(d0_ref, d1_ref, a0_ref, a1_ref, p1_ref, b1_ref, w_ref, out_ref):
        dinv = lax.rsqrt(d0_ref[...] + d1_ref[...] + 1.0)
        g = dinv * (a0_ref[...] + a1_ref[...] + p1_ref[...]) + b1_ref[...]
        g = jnp.maximum(g, 0.0)
        z = jnp.dot(
            g, w_ref[...],
            preferred_element_type=jnp.float32,
            precision=lax.Precision.HIGHEST,
        )
        out_ref[...] = z * dinv

    return pl.pallas_call(
        body,
        grid=(GRID,),
        in_specs=[
            pl.BlockSpec((RB, 1), lambda i: (i, 0)),
            pl.BlockSpec((RB, 1), lambda i: (i, 0)),
            pl.BlockSpec((RB, D), lambda i: (i, 0)),
            pl.BlockSpec((RB, D), lambda i: (i, 0)),
            pl.BlockSpec((RB, D), lambda i: (i, 0)),
            pl.BlockSpec((1, D), lambda i: (0, 0)),
            pl.BlockSpec((D, D), lambda i: (0, 0)),
        ],
        out_specs=pl.BlockSpec((RB, D), lambda i: (i, 0)),
        out_shape=jax.ShapeDtypeStruct((N, D), jnp.float32),
    )(d0, d1, a0, a1, p1, b1, W2)


def _tc_last(d0, d1, a0, a1, p2, b2):
    def body(d0_ref, d1_ref, a0_ref, a1_ref, p2_ref, b2_ref, out_ref):
        dinv = lax.rsqrt(d0_ref[...] + d1_ref[...] + 1.0)
        out_ref[...] = dinv * (a0_ref[...] + a1_ref[...] + p2_ref[...]) + b2_ref[...]

    return pl.pallas_call(
        body,
        grid=(GRID,),
        in_specs=[
            pl.BlockSpec((RB, 1), lambda i: (i, 0)),
            pl.BlockSpec((RB, 1), lambda i: (i, 0)),
            pl.BlockSpec((RB, D), lambda i: (i, 0)),
            pl.BlockSpec((RB, D), lambda i: (i, 0)),
            pl.BlockSpec((RB, D), lambda i: (i, 0)),
            pl.BlockSpec((1, D), lambda i: (0, 0)),
        ],
        out_specs=pl.BlockSpec((RB, D), lambda i: (i, 0)),
        out_shape=jax.ShapeDtypeStruct((N, D), jnp.float32),
    )(d0, d1, a0, a1, p2, b2)


def kernel(x, edge_index, W1, b1, W2, b2):
    src4 = edge_index[0].reshape(NW, IP, KP, CH)
    dst4 = edge_index[1].reshape(NW, IP, KP, CH)

    deg0, deg1 = _deg_partials(dst4)
    d0 = deg0.reshape(NPAD, 1)
    d1 = deg1.reshape(NPAD, 1)
    b1r = b1.reshape(1, D)
    b2r = b2.reshape(1, D)

    p1 = _tc_first(d0, d1, x, W1)
    a0, a1 = _aggregate(p1, src4, dst4)
    p2 = _tc_mid(d0, d1, a0, a1, p1, b1r, W2)
    c0, c1 = _aggregate(p2, src4, dst4)
    out = _tc_last(d0, d1, c0, c1, p2, b2r)
    return out


# speedup vs baseline: 1.0206x; 1.0206x over previous
import functools

import jax
import jax.numpy as jnp
from jax import lax
from jax.experimental import pallas as pl
from jax.experimental.pallas import tpu as pltpu
from jax.experimental.pallas import tpu_sc as plsc

N = 10000
E = 320000
D = 128

NC = 2
NS = 16
NW = NC * NS
NPAD = 10240
SEG = NPAD // NS
EW = E // NW
CH = 80
KC = EW // CH

IP = 5
KP = KC // IP
ZC = 80

RB = 1024
GRID = (NPAD + RB - 1) // RB


def _sc_mesh():
    return plsc.VectorSubcoreMesh(
        core_axis_name="c", subcore_axis_name="s", num_cores=NC, num_subcores=NS
    )


def _deg_partials(dst3):
    @functools.partial(
        pl.kernel,
        out_type=(
            jax.ShapeDtypeStruct((NPAD,), jnp.float32),
            jax.ShapeDtypeStruct((NPAD,), jnp.float32),
        ),
        mesh=_sc_mesh(),
        scratch_types=[
            pltpu.VMEM((IP, KP, CH), jnp.int32),
            pltpu.VMEM((CH,), jnp.float32),
            pltpu.VMEM((SEG,), jnp.float32),
            pltpu.VMEM_SHARED((NPAD,), jnp.float32),
            pltpu.SemaphoreType.DMA,
        ],
    )
    def k(dst_hbm, deg0_hbm, deg1_hbm, didx3, ones, zbuf, deg_sh, sem):
        cid = lax.axis_index("c")
        sid = lax.axis_index("s")
        wid = cid * NS + sid
        pltpu.sync_copy(dst_hbm.at[wid], didx3)
        for i in range(CH // 16):
            ones[pl.ds(i * 16, 16)] = jnp.ones((16,), jnp.float32)

        def zfill(i, _):
            zbuf[pl.ds(i * 16, 16)] = jnp.zeros((16,), jnp.float32)
            return 0

        lax.fori_loop(0, SEG // 16, zfill, 0)
        pltpu.sync_copy(zbuf, deg_sh.at[pl.ds(sid * SEG, SEG)])
        plsc.subcore_barrier()

        def issue(kk, _):
            pltpu.async_copy(ones, deg_sh.at[didx3.at[kk // KP, kk % KP]], sem, add=True)
            return 0

        lax.fori_loop(0, KC, issue, 0)

        def drain(kk, _):
            pltpu.make_async_copy(ones, deg_sh.at[didx3.at[kk // KP, kk % KP]], sem).wait()
            return 0

        lax.fori_loop(0, KC, drain, 0)
        plsc.subcore_barrier()

        @pl.when(cid == 0)
        def _():
            pltpu.sync_copy(
                deg_sh.at[pl.ds(sid * SEG, SEG)], deg0_hbm.at[pl.ds(sid * SEG, SEG)]
            )

        @pl.when(cid == 1)
        def _():
            pltpu.sync_copy(
                deg_sh.at[pl.ds(sid * SEG, SEG)], deg1_hbm.at[pl.ds(sid * SEG, SEG)]
            )

    return k(dst3)


def _aggregate(p, src3, dst3):
    @functools.partial(
        pl.kernel,
        out_type=(
            jax.ShapeDtypeStruct((NPAD, D), jnp.float32),
            jax.ShapeDtypeStruct((NPAD, D), jnp.float32),
        ),
        mesh=_sc_mesh(),
        scratch_types=[
            pltpu.VMEM((KP, CH), jnp.int32),
            pltpu.VMEM((KP, CH), jnp.int32),
            [pltpu.VMEM((CH, D), jnp.float32)] * 3,
            pltpu.VMEM_SHARED((NPAD, D), jnp.float32),
            [pltpu.SemaphoreType.DMA] * 3,
            [pltpu.SemaphoreType.DMA] * 3,
            pltpu.SemaphoreType.DMA,
        ],
    )
    def k(p_hbm, src_hbm, dst_hbm, a0_hbm, a1_hbm, sidx2, didx2,
          bufs, acc_sh, gs, ss, zsem):
        cid = lax.axis_index("c")
        sid = lax.axis_index("s")
        wid = cid * NS + sid
        buf0 = bufs[0]

        def zfill(i, _):
            for c in range(D // 16):
                buf0[i, pl.ds(c * 16, 16)] = jnp.zeros((16,), jnp.float32)
            return 0

        lax.fori_loop(0, CH, zfill, 0)
        for i in range(SEG // ZC):
            pltpu.async_copy(
                buf0.at[pl.ds(0, ZC)], acc_sh.at[pl.ds(sid * SEG + i * ZC, ZC)], zsem)
        for i in range(SEG // ZC):
            pltpu.make_async_copy(
                buf0.at[pl.ds(0, ZC)], acc_sh.at[pl.ds(sid * SEG + i * ZC, ZC)], zsem).wait()
        plsc.subcore_barrier()

        def issue_g(kk, r):
            pltpu.async_copy(p_hbm.at[sidx2.at[kk]], bufs[r], gs[r])

        def drain_g(kk, r):
            pltpu.make_async_copy(p_hbm.at[sidx2.at[kk]], bufs[r], gs[r]).wait()

        def issue_s(kk, r):
            pltpu.async_copy(bufs[r], acc_sh.at[didx2.at[kk]], ss[r], add=True)

        def drain_s(kk, r):
            pltpu.make_async_copy(bufs[r], acc_sh.at[didx2.at[kk]], ss[r]).wait()

        def full_step(kk, r):
            drain_g(kk, r)
            issue_s(kk, r)
            drain_s(kk - 1, (r + 2) % 3)
            issue_g(kk + 2, (r + 2) % 3)

        for p in range(IP):
            pltpu.sync_copy(src_hbm.at[wid, p], sidx2)
            pltpu.sync_copy(dst_hbm.at[wid, p], didx2)

            issue_g(0, 0)
            issue_g(1, 1)
            drain_g(0, 0)
            issue_s(0, 0)
            issue_g(2, 2)
            full_step(1, 1)
            full_step(2, 2)

            def trio(j, _):
                kk = 3 + 3 * j
                full_step(kk, 0)
                full_step(kk + 1, 1)
                full_step(kk + 2, 2)
                return 0

            lax.fori_loop(0, (KP - 7) // 3, trio, 0)

            full_step(KP - 4, 0)
            full_step(KP - 3, 1)
            drain_g(KP - 2, 2)
            issue_s(KP - 2, 2)
            drain_s(KP - 3, 1)
            drain_g(KP - 1, 0)
            issue_s(KP - 1, 0)
            drain_s(KP - 2, 2)
            drain_s(KP - 1, 0)
        plsc.subcore_barrier()

        @pl.when(cid == 0)
        def _():
            pltpu.sync_copy(
                acc_sh.at[pl.ds(sid * SEG, SEG)], a0_hbm.at[pl.ds(sid * SEG, SEG)]
            )

        @pl.when(cid == 1)
        def _():
            pltpu.sync_copy(
                acc_sh.at[pl.ds(sid * SEG, SEG)], a1_hbm.at[pl.ds(sid * SEG, SEG)]
            )

    return k(p, src3, dst3)


def _tc_first(d0, d1, x, W1):
    def body(d0_ref, d1_ref, x_ref, w_ref, p_ref):
        dinv = lax.rsqrt(d0_ref[...] + d1_ref[...] + 1.0)
        z = jnp.dot(
            x_ref[...], w_ref[...],
            preferred_element_type=jnp.float32,
            precision=lax.Precision.HIGHEST,
        )
        p_ref[...] = z * dinv

    return pl.pallas_call(
        body,
        grid=(GRID,),
        in_specs=[
            pl.BlockSpec((RB, 1), lambda i: (i, 0)),
            pl.BlockSpec((RB, 1), lambda i: (i, 0)),
            pl.BlockSpec((RB, D), lambda i: (i, 0)),
            pl.BlockSpec((D, D), lambda i: (0, 0)),
        ],
        out_specs=pl.BlockSpec((RB, D), lambda i: (i, 0)),
        out_shape=jax.ShapeDtypeStruct((N, D), jnp.float32),
    )(d0, d1, x, W1)


def _tc_mid(d0, d1, a0, a1, p1, b1, W2):
    def body(d0_ref, d1_ref, a0_ref, a1_ref, p1_ref, b1_ref, w_ref, out_ref):
        dinv = lax.rsqrt(d0_ref[...] + d1_ref[...] + 1.0)
        g = jnp.maximum(
            dinv * (a0_ref[...] + a1_ref[...] + p1_ref[...]) + b1_ref[...], 0.0)
        z = jnp.dot(
            g, w_ref[...],
            preferred_element_type=jnp.float32,
            precision=lax.Precision.HIGHEST,
        )
        out_ref[...] = z * dinv

    return pl.pallas_call(
        body,
        grid=(GRID,),
        in_specs=[
            pl.BlockSpec((RB, 1), lambda i: (i, 0)),
            pl.BlockSpec((RB, 1), lambda i: (i, 0)),
            pl.BlockSpec((RB, D), lambda i: (i, 0)),
            pl.BlockSpec((RB, D), lambda i: (i, 0)),
            pl.BlockSpec((RB, D), lambda i: (i, 0)),
            pl.BlockSpec((1, D), lambda i: (0, 0)),
            pl.BlockSpec((D, D), lambda i: (0, 0)),
        ],
        out_specs=pl.BlockSpec((RB, D), lambda i: (i, 0)),
        out_shape=jax.ShapeDtypeStruct((N, D), jnp.float32),
    )(d0, d1, a0, a1, p1, b1, W2)


def _tc_last(d0, d1, a0, a1, p2, b2):
    def body(d0_ref, d1_ref, a0_ref, a1_ref, p2_ref, b2_ref, out_ref):
        dinv = lax.rsqrt(d0_ref[...] + d1_ref[...] + 1.0)
        out_ref[...] = (dinv * (a0_ref[...] + a1_ref[...] + p2_ref[...])
                        + b2_ref[...])

    return pl.pallas_call(
        body,
        grid=(GRID,),
        in_specs=[
            pl.BlockSpec((RB, 1), lambda i: (i, 0)),
            pl.BlockSpec((RB, 1), lambda i: (i, 0)),
            pl.BlockSpec((RB, D), lambda i: (i, 0)),
            pl.BlockSpec((RB, D), lambda i: (i, 0)),
            pl.BlockSpec((RB, D), lambda i: (i, 0)),
            pl.BlockSpec((1, D), lambda i: (0, 0)),
        ],
        out_specs=pl.BlockSpec((RB, D), lambda i: (i, 0)),
        out_shape=jax.ShapeDtypeStruct((N, D), jnp.float32),
    )(d0, d1, a0, a1, p2, b2)


def kernel(x, edge_index, W1, b1, W2, b2):
    src4 = edge_index[0].reshape(NW, IP, KP, CH)
    dst4 = edge_index[1].reshape(NW, IP, KP, CH)

    deg0, deg1 = _deg_partials(dst4)
    d0 = deg0.reshape(NPAD, 1)
    d1 = deg1.reshape(NPAD, 1)
    b1r = b1.reshape(1, D)
    b2r = b2.reshape(1, D)

    p1 = _tc_first(d0, d1, x, W1)
    a0, a1 = _aggregate(p1, src4, dst4)
    p2 = _tc_mid(d0, d1, a0, a1, p1, b1r, W2)
    c0, c1 = _aggregate(p2, src4, dst4)
    out = _tc_last(d0, d1, c0, c1, p2, b2r)
    return out


# speedup vs baseline: 1.0214x; 1.0008x over previous
import functools

import jax
import jax.numpy as jnp
from jax import lax
from jax.experimental import pallas as pl
from jax.experimental.pallas import tpu as pltpu
from jax.experimental.pallas import tpu_sc as plsc

N = 10000
E = 320000
D = 128

NC = 2
NS = 16
NW = NC * NS
NPAD = 10240
SEG = NPAD // NS
EW = E // NW
CH = 80
KC = EW // CH

IP = 5
KP = KC // IP
ZC = 80

RB = 1024
GRID = (NPAD + RB - 1) // RB


def _sc_mesh():
    return plsc.VectorSubcoreMesh(
        core_axis_name="c", subcore_axis_name="s", num_cores=NC, num_subcores=NS
    )


def _deg_partials(dst3):
    @functools.partial(
        pl.kernel,
        out_type=(
            jax.ShapeDtypeStruct((NPAD,), jnp.float32),
            jax.ShapeDtypeStruct((NPAD,), jnp.float32),
        ),
        mesh=_sc_mesh(),
        scratch_types=[
            pltpu.VMEM((IP, KP, CH), jnp.int32),
            pltpu.VMEM((CH,), jnp.float32),
            pltpu.VMEM((SEG,), jnp.float32),
            pltpu.VMEM_SHARED((NPAD,), jnp.float32),
            pltpu.SemaphoreType.DMA,
        ],
    )
    def k(dst_hbm, deg0_hbm, deg1_hbm, didx3, ones, zbuf, deg_sh, sem):
        cid = lax.axis_index("c")
        sid = lax.axis_index("s")
        wid = cid * NS + sid
        pltpu.sync_copy(dst_hbm.at[wid], didx3)
        for i in range(CH // 16):
            ones[pl.ds(i * 16, 16)] = jnp.ones((16,), jnp.float32)

        def zfill(i, _):
            zbuf[pl.ds(i * 16, 16)] = jnp.zeros((16,), jnp.float32)
            return 0

        lax.fori_loop(0, SEG // 16, zfill, 0)
        pltpu.sync_copy(zbuf, deg_sh.at[pl.ds(sid * SEG, SEG)])
        plsc.subcore_barrier()

        def issue(kk, _):
            pltpu.async_copy(ones, deg_sh.at[didx3.at[kk // KP, kk % KP]], sem, add=True)
            return 0

        lax.fori_loop(0, KC, issue, 0)

        def drain(kk, _):
            pltpu.make_async_copy(ones, deg_sh.at[didx3.at[kk // KP, kk % KP]], sem).wait()
            return 0

        lax.fori_loop(0, KC, drain, 0)
        plsc.subcore_barrier()

        @pl.when(cid == 0)
        def _():
            pltpu.sync_copy(
                deg_sh.at[pl.ds(sid * SEG, SEG)], deg0_hbm.at[pl.ds(sid * SEG, SEG)]
            )

        @pl.when(cid == 1)
        def _():
            pltpu.sync_copy(
                deg_sh.at[pl.ds(sid * SEG, SEG)], deg1_hbm.at[pl.ds(sid * SEG, SEG)]
            )

    return k(dst3)


def _aggregate(p, src3, dst3):
    @functools.partial(
        pl.kernel,
        out_type=(
            jax.ShapeDtypeStruct((NPAD, D), jnp.float32),
            jax.ShapeDtypeStruct((NPAD, D), jnp.float32),
        ),
        mesh=_sc_mesh(),
        scratch_types=[
            pltpu.VMEM((KP, CH), jnp.int32),
            pltpu.VMEM((KP, CH), jnp.int32),
            [pltpu.VMEM((CH, D), jnp.float32)] * 3,
            pltpu.VMEM_SHARED((NPAD, D), jnp.float32),
            [pltpu.SemaphoreType.DMA] * 3,
            [pltpu.SemaphoreType.DMA] * 3,
            pltpu.SemaphoreType.DMA,
        ],
    )
    def k(p_hbm, src_hbm, dst_hbm, a0_hbm, a1_hbm, sidx2, didx2,
          bufs, acc_sh, gs, ss, zsem):
        cid = lax.axis_index("c")
        sid = lax.axis_index("s")
        wid = cid * NS + sid
        buf0 = bufs[0]

        def zfill(i, _):
            for c in range(D // 16):
                buf0[i, pl.ds(c * 16, 16)] = jnp.zeros((16,), jnp.float32)
            return 0

        lax.fori_loop(0, CH, zfill, 0)
        for i in range(SEG // ZC):
            pltpu.async_copy(
                buf0.at[pl.ds(0, ZC)], acc_sh.at[pl.ds(sid * SEG + i * ZC, ZC)], zsem)
        for i in range(SEG // ZC):
            pltpu.make_async_copy(
                buf0.at[pl.ds(0, ZC)], acc_sh.at[pl.ds(sid * SEG + i * ZC, ZC)], zsem).wait()
        plsc.subcore_barrier()

        HH = CH // 2

        def issue_g(kk, r):
            pltpu.async_copy(
                p_hbm.at[sidx2.at[kk, pl.ds(0, HH)]], bufs[r].at[pl.ds(0, HH)], gs[r])
            pltpu.async_copy(
                p_hbm.at[sidx2.at[kk, pl.ds(HH, HH)]], bufs[r].at[pl.ds(HH, HH)], gs[r])

        def drain_g(kk, r):
            pltpu.make_async_copy(
                p_hbm.at[sidx2.at[kk, pl.ds(0, HH)]], bufs[r].at[pl.ds(0, HH)], gs[r]).wait()
            pltpu.make_async_copy(
                p_hbm.at[sidx2.at[kk, pl.ds(HH, HH)]], bufs[r].at[pl.ds(HH, HH)], gs[r]).wait()

        def issue_s(kk, r):
            pltpu.async_copy(bufs[r], acc_sh.at[didx2.at[kk]], ss[r], add=True)

        def drain_s(kk, r):
            pltpu.make_async_copy(bufs[r], acc_sh.at[didx2.at[kk]], ss[r]).wait()

        def full_step(kk, r):
            drain_g(kk, r)
            issue_s(kk, r)
            drain_s(kk - 1, (r + 2) % 3)
            issue_g(kk + 2, (r + 2) % 3)

        for p in range(IP):
            pltpu.sync_copy(src_hbm.at[wid, p], sidx2)
            pltpu.sync_copy(dst_hbm.at[wid, p], didx2)

            issue_g(0, 0)
            issue_g(1, 1)
            drain_g(0, 0)
            issue_s(0, 0)
            issue_g(2, 2)
            full_step(1, 1)
            full_step(2, 2)

            def trio(j, _):
                kk = 3 + 3 * j
                full_step(kk, 0)
                full_step(kk + 1, 1)
                full_step(kk + 2, 2)
                return 0

            lax.fori_loop(0, (KP - 7) // 3, trio, 0)

            full_step(KP - 4, 0)
            full_step(KP - 3, 1)
            drain_g(KP - 2, 2)
            issue_s(KP - 2, 2)
            drain_s(KP - 3, 1)
            drain_g(KP - 1, 0)
            issue_s(KP - 1, 0)
            drain_s(KP - 2, 2)
            drain_s(KP - 1, 0)
        plsc.subcore_barrier()

        @pl.when(cid == 0)
        def _():
            pltpu.sync_copy(
                acc_sh.at[pl.ds(sid * SEG, SEG)], a0_hbm.at[pl.ds(sid * SEG, SEG)]
            )

        @pl.when(cid == 1)
        def _():
            pltpu.sync_copy(
                acc_sh.at[pl.ds(sid * SEG, SEG)], a1_hbm.at[pl.ds(sid * SEG, SEG)]
            )

    return k(p, src3, dst3)


def _tc_first(d0, d1, x, W1):
    def body(d0_ref, d1_ref, x_ref, w_ref, p_ref):
        dinv = lax.rsqrt(d0_ref[...] + d1_ref[...] + 1.0)
        z = jnp.dot(
            x_ref[...], w_ref[...],
            preferred_element_type=jnp.float32,
            precision=lax.Precision.HIGHEST,
        )
        p_ref[...] = z * dinv

    return pl.pallas_call(
        body,
        grid=(GRID,),
        in_specs=[
            pl.BlockSpec((RB, 1), lambda i: (i, 0)),
            pl.BlockSpec((RB, 1), lambda i: (i, 0)),
            pl.BlockSpec((RB, D), lambda i: (i, 0)),
            pl.BlockSpec((D, D), lambda i: (0, 0)),
        ],
        out_specs=pl.BlockSpec((RB, D), lambda i: (i, 0)),
        out_shape=jax.ShapeDtypeStruct((N, D), jnp.float32),
    )(d0, d1, x, W1)


def _tc_mid(d0, d1, a0, a1, p1, b1, W2):
    def body(d0_ref, d1_ref, a0_ref, a1_ref, p1_ref, b1_ref, w_ref, out_ref):
        dinv = lax.rsqrt(d0_ref[...] + d1_ref[...] + 1.0)
        g = jnp.maximum(
            dinv * (a0_ref[...] + a1_ref[...] + p1_ref[...]) + b1_ref[...], 0.0)
        z = jnp.dot(
            g, w_ref[...],
            preferred_element_type=jnp.float32,
            precision=lax.Precision.HIGHEST,
        )
        out_ref[...] = z * dinv

    return pl.pallas_call(
        body,
        grid=(GRID,),
        in_specs=[
            pl.BlockSpec((RB, 1), lambda i: (i, 0)),
            pl.BlockSpec((RB, 1), lambda i: (i, 0)),
            pl.BlockSpec((RB, D), lambda i: (i, 0)),
            pl.BlockSpec((RB, D), lambda i: (i, 0)),
            pl.BlockSpec((RB, D), lambda i: (i, 0)),
            pl.BlockSpec((1, D), lambda i: (0, 0)),
            pl.BlockSpec((D, D), lambda i: (0, 0)),
        ],
        out_specs=pl.BlockSpec((RB, D), lambda i: (i, 0)),
        out_shape=jax.ShapeDtypeStruct((N, D), jnp.float32),
    )(d0, d1, a0, a1, p1, b1, W2)


def _tc_last(d0, d1, a0, a1, p2, b2):
    def body(d0_ref, d1_ref, a0_ref, a1_ref, p2_ref, b2_ref, out_ref):
        dinv = lax.rsqrt(d0_ref[...] + d1_ref[...] + 1.0)
        out_ref[...] = (dinv * (a0_ref[...] + a1_ref[...] + p2_ref[...])
                        + b2_ref[...])

    return pl.pallas_call(
        body,
        grid=(GRID,),
        in_specs=[
            pl.BlockSpec((RB, 1), lambda i: (i, 0)),
            pl.BlockSpec((RB, 1), lambda i: (i, 0)),
            pl.BlockSpec((RB, D), lambda i: (i, 0)),
            pl.BlockSpec((RB, D), lambda i: (i, 0)),
            pl.BlockSpec((RB, D), lambda i: (i, 0)),
            pl.BlockSpec((1, D), lambda i: (0, 0)),
        ],
        out_specs=pl.BlockSpec((RB, D), lambda i: (i, 0)),
        out_shape=jax.ShapeDtypeStruct((N, D), jnp.float32),
    )(d0, d1, a0, a1, p2, b2)


def kernel(x, edge_index, W1, b1, W2, b2):
    src4 = edge_index[0].reshape(NW, IP, KP, CH)
    dst4 = edge_index[1].reshape(NW, IP, KP, CH)

    deg0, deg1 = _deg_partials(dst4)
    d0 = deg0.reshape(NPAD, 1)
    d1 = deg1.reshape(NPAD, 1)
    b1r = b1.reshape(1, D)
    b2r = b2.reshape(1, D)

    p1 = _tc_first(d0, d1, x, W1)
    a0, a1 = _aggregate(p1, src4, dst4)
    p2 = _tc_mid(d0, d1, a0, a1, p1, b1r, W2)
    c0, c1 = _aggregate(p2, src4, dst4)
    out = _tc_last(d0, d1, c0, c1, p2, b2r)
    return out


# speedup vs baseline: 1.0328x; 1.0111x over previous
import functools

import jax
import jax.numpy as jnp
from jax import lax
from jax.experimental import pallas as pl
from jax.experimental.pallas import tpu as pltpu
from jax.experimental.pallas import tpu_sc as plsc

N = 10000
E = 320000
D = 128

NC = 2
NS = 16
NW = NC * NS
NPAD = 10240
SEG = NPAD // NS
EW = E // NW
CH = 80
KC = EW // CH

IP = 5
KP = KC // IP
ZC = 80

RB = 1024
GRID = (NPAD + RB - 1) // RB


def _sc_mesh():
    return plsc.VectorSubcoreMesh(
        core_axis_name="c", subcore_axis_name="s", num_cores=NC, num_subcores=NS
    )


def _deg_partials(dst3):
    @functools.partial(
        pl.kernel,
        out_type=(
            jax.ShapeDtypeStruct((NPAD,), jnp.float32),
            jax.ShapeDtypeStruct((NPAD,), jnp.float32),
        ),
        mesh=_sc_mesh(),
        scratch_types=[
            pltpu.VMEM((IP, KP, CH), jnp.int32),
            pltpu.VMEM((CH,), jnp.float32),
            pltpu.VMEM((SEG,), jnp.float32),
            pltpu.VMEM_SHARED((NPAD,), jnp.float32),
            pltpu.SemaphoreType.DMA,
        ],
    )
    def k(dst_hbm, deg0_hbm, deg1_hbm, didx3, ones, zbuf, deg_sh, sem):
        cid = lax.axis_index("c")
        sid = lax.axis_index("s")
        wid = cid * NS + sid
        pltpu.sync_copy(dst_hbm.at[wid], didx3)
        for i in range(CH // 16):
            ones[pl.ds(i * 16, 16)] = jnp.ones((16,), jnp.float32)

        def zfill(i, _):
            zbuf[pl.ds(i * 16, 16)] = jnp.zeros((16,), jnp.float32)
            return 0

        lax.fori_loop(0, SEG // 16, zfill, 0)
        pltpu.sync_copy(zbuf, deg_sh.at[pl.ds(sid * SEG, SEG)])
        plsc.subcore_barrier()

        def issue(kk, _):
            pltpu.async_copy(ones, deg_sh.at[didx3.at[kk // KP, kk % KP]], sem, add=True)
            return 0

        lax.fori_loop(0, KC, issue, 0)

        def drain(kk, _):
            pltpu.make_async_copy(ones, deg_sh.at[didx3.at[kk // KP, kk % KP]], sem).wait()
            return 0

        lax.fori_loop(0, KC, drain, 0)
        plsc.subcore_barrier()

        @pl.when(cid == 0)
        def _():
            pltpu.sync_copy(
                deg_sh.at[pl.ds(sid * SEG, SEG)], deg0_hbm.at[pl.ds(sid * SEG, SEG)]
            )

        @pl.when(cid == 1)
        def _():
            pltpu.sync_copy(
                deg_sh.at[pl.ds(sid * SEG, SEG)], deg1_hbm.at[pl.ds(sid * SEG, SEG)]
            )

    return k(dst3)


def _aggregate(p, src3, dst3):
    @functools.partial(
        pl.kernel,
        out_type=(
            jax.ShapeDtypeStruct((NPAD, D), jnp.float32),
            jax.ShapeDtypeStruct((NPAD, D), jnp.float32),
        ),
        mesh=_sc_mesh(),
        scratch_types=[
            pltpu.VMEM((KP, CH), jnp.int32),
            pltpu.VMEM((KP, CH), jnp.int32),
            [pltpu.VMEM((CH, D), jnp.float32)] * 3,
            pltpu.VMEM_SHARED((NPAD, D), jnp.float32),
            [pltpu.SemaphoreType.DMA] * 3,
            [pltpu.SemaphoreType.DMA] * 3,
            pltpu.SemaphoreType.DMA,
        ],
    )
    def k(p_hbm, src_hbm, dst_hbm, a0_hbm, a1_hbm, sidx2, didx2,
          bufs, acc_sh, gs, ss, zsem):
        cid = lax.axis_index("c")
        sid = lax.axis_index("s")
        wid = cid * NS + sid
        buf0 = bufs[0]

        def zfill(i, _):
            for c in range(D // 16):
                buf0[i, pl.ds(c * 16, 16)] = jnp.zeros((16,), jnp.float32)
            return 0

        lax.fori_loop(0, CH, zfill, 0)
        for i in range(SEG // ZC):
            pltpu.async_copy(
                buf0.at[pl.ds(0, ZC)], acc_sh.at[pl.ds(sid * SEG + i * ZC, ZC)], zsem)
        for i in range(SEG // ZC):
            pltpu.make_async_copy(
                buf0.at[pl.ds(0, ZC)], acc_sh.at[pl.ds(sid * SEG + i * ZC, ZC)], zsem).wait()
        plsc.subcore_barrier()

        def issue_g(kk, r):
            pltpu.async_copy(p_hbm.at[sidx2.at[kk]], bufs[r], gs[r])

        def drain_g(kk, r):
            pltpu.make_async_copy(p_hbm.at[sidx2.at[kk]], bufs[r], gs[r]).wait()

        def issue_s(kk, r):
            pltpu.async_copy(bufs[r], acc_sh.at[didx2.at[kk]], ss[r], add=True)

        def drain_s(kk, r):
            pltpu.make_async_copy(bufs[r], acc_sh.at[didx2.at[kk]], ss[r]).wait()

        def full_step(kk, r):
            drain_g(kk, r)
            issue_s(kk, r)
            drain_s(kk - 1, (r + 2) % 3)
            issue_g(kk + 2, (r + 2) % 3)

        for p in range(IP):
            pltpu.sync_copy(src_hbm.at[wid, p], sidx2)
            pltpu.sync_copy(dst_hbm.at[wid, p], didx2)

            issue_g(0, 0)
            issue_g(1, 1)
            drain_g(0, 0)
            issue_s(0, 0)
            issue_g(2, 2)
            full_step(1, 1)
            full_step(2, 2)

            def trio(j, _):
                kk = 3 + 3 * j
                full_step(kk, 0)
                full_step(kk + 1, 1)
                full_step(kk + 2, 2)
                return 0

            lax.fori_loop(0, (KP - 7) // 3, trio, 0)

            full_step(KP - 4, 0)
            full_step(KP - 3, 1)
            drain_g(KP - 2, 2)
            issue_s(KP - 2, 2)
            drain_s(KP - 3, 1)
            drain_g(KP - 1, 0)
            issue_s(KP - 1, 0)
            drain_s(KP - 2, 2)
            drain_s(KP - 1, 0)
        plsc.subcore_barrier()

        @pl.when(cid == 0)
        def _():
            pltpu.sync_copy(
                acc_sh.at[pl.ds(sid * SEG, SEG)], a0_hbm.at[pl.ds(sid * SEG, SEG)]
            )

        @pl.when(cid == 1)
        def _():
            pltpu.sync_copy(
                acc_sh.at[pl.ds(sid * SEG, SEG)], a1_hbm.at[pl.ds(sid * SEG, SEG)]
            )

    return k(p, src3, dst3)


def _tc_first(d0, d1, x, W1):
    def body(d0_ref, d1_ref, x_ref, w_ref, p_ref):
        dinv = lax.rsqrt(d0_ref[...] + d1_ref[...] + 1.0)
        z = jnp.dot(
            x_ref[...], w_ref[...],
            preferred_element_type=jnp.float32,
        )
        p_ref[...] = z * dinv

    return pl.pallas_call(
        body,
        grid=(GRID,),
        in_specs=[
            pl.BlockSpec((RB, 1), lambda i: (i, 0)),
            pl.BlockSpec((RB, 1), lambda i: (i, 0)),
            pl.BlockSpec((RB, D), lambda i: (i, 0)),
            pl.BlockSpec((D, D), lambda i: (0, 0)),
        ],
        out_specs=pl.BlockSpec((RB, D), lambda i: (i, 0)),
        out_shape=jax.ShapeDtypeStruct((N, D), jnp.float32),
    )(d0, d1, x, W1)


def _tc_mid(d0, d1, a0, a1, p1, b1, W2):
    def body(d0_ref, d1_ref, a0_ref, a1_ref, p1_ref, b1_ref, w_ref, out_ref):
        dinv = lax.rsqrt(d0_ref[...] + d1_ref[...] + 1.0)
        g = jnp.maximum(
            dinv * (a0_ref[...] + a1_ref[...] + p1_ref[...]) + b1_ref[...], 0.0)
        z = jnp.dot(
            g, w_ref[...],
            preferred_element_type=jnp.float32,
        )
        out_ref[...] = z * dinv

    return pl.pallas_call(
        body,
        grid=(GRID,),
        in_specs=[
            pl.BlockSpec((RB, 1), lambda i: (i, 0)),
            pl.BlockSpec((RB, 1), lambda i: (i, 0)),
            pl.BlockSpec((RB, D), lambda i: (i, 0)),
            pl.BlockSpec((RB, D), lambda i: (i, 0)),
            pl.BlockSpec((RB, D), lambda i: (i, 0)),
            pl.BlockSpec((1, D), lambda i: (0, 0)),
            pl.BlockSpec((D, D), lambda i: (0, 0)),
        ],
        out_specs=pl.BlockSpec((RB, D), lambda i: (i, 0)),
        out_shape=jax.ShapeDtypeStruct((N, D), jnp.float32),
    )(d0, d1, a0, a1, p1, b1, W2)


def _tc_last(d0, d1, a0, a1, p2, b2):
    def body(d0_ref, d1_ref, a0_ref, a1_ref, p2_ref, b2_ref, out_ref):
        dinv = lax.rsqrt(d0_ref[...] + d1_ref[...] + 1.0)
        out_ref[...] = (dinv * (a0_ref[...] + a1_ref[...] + p2_ref[...])
                        + b2_ref[...])

    return pl.pallas_call(
        body,
        grid=(GRID,),
        in_specs=[
            pl.BlockSpec((RB, 1), lambda i: (i, 0)),
            pl.BlockSpec((RB, 1), lambda i: (i, 0)),
            pl.BlockSpec((RB, D), lambda i: (i, 0)),
            pl.BlockSpec((RB, D), lambda i: (i, 0)),
            pl.BlockSpec((RB, D), lambda i: (i, 0)),
            pl.BlockSpec((1, D), lambda i: (0, 0)),
        ],
        out_specs=pl.BlockSpec((RB, D), lambda i: (i, 0)),
        out_shape=jax.ShapeDtypeStruct((N, D), jnp.float32),
    )(d0, d1, a0, a1, p2, b2)


def kernel(x, edge_index, W1, b1, W2, b2):
    src4 = edge_index[0].reshape(NW, IP, KP, CH)
    dst4 = edge_index[1].reshape(NW, IP, KP, CH)

    deg0, deg1 = _deg_partials(dst4)
    d0 = deg0.reshape(NPAD, 1)
    d1 = deg1.reshape(NPAD, 1)
    b1r = b1.reshape(1, D)
    b2r = b2.reshape(1, D)

    p1 = _tc_first(d0, d1, x, W1)
    a0, a1 = _aggregate(p1, src4, dst4)
    p2 = _tc_mid(d0, d1, a0, a1, p1, b1r, W2)
    c0, c1 = _aggregate(p2, src4, dst4)
    out = _tc_last(d0, d1, c0, c1, p2, b2r)
    return out
